# baseline (speedup 1.0000x reference)
import functools

import jax
import jax.numpy as jnp
from jax import lax
from jax.experimental import pallas as pl
from jax.experimental.pallas import tpu as pltpu
from jax.experimental.pallas import tpu_sc as plsc

_D = 64
_NC, _NS = 2, 16
_NW = _NC * _NS
_CHUNK = 128
_TBLK = 1024
_TILE = 2048
_EPS = 1e-5
_F32 = jnp.float32
_HI = lax.Precision.HIGHEST


def _transcat_body(ga, ma, out_ref):
    tg = jnp.transpose(ga[...].astype(jnp.bfloat16), (1, 0))
    tm = jnp.transpose(ma[...].astype(jnp.bfloat16), (1, 0))
    lo = lax.bitcast_convert_type(tg, jnp.uint16).astype(jnp.uint32)
    hi = lax.bitcast_convert_type(tm, jnp.uint16).astype(jnp.uint32)
    out_ref[...] = lax.bitcast_convert_type(lo | (hi << 16), _F32)


def _transcat(gt, mt):
    V = gt.shape[1]
    n = -(-V // _TBLK)
    return pl.pallas_call(
        _transcat_body,
        grid=(n,),
        in_specs=[pl.BlockSpec((_D, _TBLK), lambda i: (0, i)),
                  pl.BlockSpec((_D, _TBLK), lambda i: (0, i))],
        out_specs=pl.BlockSpec((_TBLK, _D), lambda i: (i, 0)),
        out_shape=jax.ShapeDtypeStruct((V, _D), _F32),
    )(gt, mt)


def _sc_gather_body(uidx, iidx, ucat, icat, out_u, out_i,
                    idxu_v, idxi_v, b0, b1, b2, b3, s0, s1, s2, s3):
    rpw = idxu_v.shape[0]
    cpt = rpw // _CHUNK
    wid = lax.axis_index("s") * _NC + lax.axis_index("c")
    base = wid * rpw
    pltpu.sync_copy(uidx.at[pl.ds(base, rpw)], idxu_v)
    pltpu.sync_copy(iidx.at[pl.ds(base, rpw)], idxi_v)

    bufs = (b0, b1, b2, b3)
    sems = (s0, s1, s2, s3)
    units = ([(ucat, idxu_v, out_u, j) for j in range(cpt)]
             + [(icat, idxi_v, out_i, j) for j in range(cpt)])
    nbuf = len(bufs)
    copies = {}

    def fire(t):
        tab, idxv, _, j = units[t]
        copies[t] = pltpu.async_copy(
            tab.at[idxv.at[pl.ds(j * _CHUNK, _CHUNK)]],
            bufs[t % nbuf], sems[t % nbuf])

    def drain(t):
        copies[t].wait()
        _, _, out, j = units[t]
        pltpu.sync_copy(bufs[t % nbuf], out.at[pl.ds(base + j * _CHUNK, _CHUNK)])

    for t in range(len(units)):
        if t >= nbuf:
            drain(t - nbuf)
        fire(t)
    for t in range(len(units) - nbuf, len(units)):
        drain(t)


def _accum_stats(i, y, st_ref):
    ps = jnp.sum(y, axis=0, keepdims=True)
    pq = jnp.sum(y * y, axis=0, keepdims=True)
    part = jnp.concatenate([ps, pq], axis=0)

    @pl.when(i == 0)
    def _():
        st_ref[...] = part

    @pl.when(i > 0)
    def _():
        st_ref[...] += part


def _bn_relu_from_stats(st_ref, n, g, be, y):
    mean = st_ref[0:1, :] * (1.0 / n)
    var = st_ref[1:2, :] * (1.0 / n) - mean * mean
    return jnp.maximum((y - mean) * lax.rsqrt(var + _EPS) * g + be, 0.0)


def _unpack_pair(w):
    u = lax.bitcast_convert_type(w, jnp.uint32)
    g = lax.bitcast_convert_type((u & 0xFFFF).astype(jnp.uint16), jnp.bfloat16)
    m = lax.bitcast_convert_type((u >> 16).astype(jnp.uint16), jnp.bfloat16)
    return g, m


def _stage1_body(uc, ic, w1ut, w1it, b1, y1_ref, st1_ref, xg_ref):
    i = pl.program_id(0)
    ug, um = _unpack_pair(uc[...])
    ig, im = _unpack_pair(ic[...])
    y1 = (jnp.dot(um, w1ut[...], preferred_element_type=_F32)
          + jnp.dot(im, w1it[...], preferred_element_type=_F32)
          + b1[...])
    y1_ref[...] = y1
    xg_ref[...] = ug.astype(_F32) * ig.astype(_F32)
    _accum_stats(i, y1, st1_ref)


def _stage2_body(y1, st1, g1, be1, w2t, b2, y2_ref, st2_ref, *, n):
    i = pl.program_id(0)
    h1 = _bn_relu_from_stats(st1, n, g1[...], be1[...], y1[...])
    y2 = jnp.dot(h1, w2t[...], preferred_element_type=_F32, precision=_HI) + b2[...]
    y2_ref[...] = y2
    _accum_stats(i, y2, st2_ref)


def _stage3_body(y2, st2, g2, be2, xg, wg, wh, bo, out_ref, *, n):
    h2 = _bn_relu_from_stats(st2, n, g2[...], be2[...], y2[...])
    s = (jnp.sum(xg[...] * wg[...], axis=1, keepdims=True)
         + jnp.sum(h2 * wh[...], axis=1, keepdims=True) + bo[...])
    out_ref[...] = 1.0 / (1.0 + jnp.exp(-s))


def kernel(user_idx, item_idx, gmf_user, gmf_item, mlp_user, mlp_item,
           W1, b1, g1, be1, W2, b2, g2, be2, Wout, bout):
    B = user_idx.shape[0]
    uidx = user_idx.astype(jnp.int32)
    iidx = item_idx.astype(jnp.int32)

    ucat = _transcat(gmf_user.T, mlp_user.T)
    icat = _transcat(gmf_item.T, mlp_item.T)

    rpw = B // _NW
    sc_gather = pl.kernel(
        _sc_gather_body,
        out_type=(jax.ShapeDtypeStruct((B, _D), _F32),) * 2,
        mesh=plsc.VectorSubcoreMesh(core_axis_name="c", subcore_axis_name="s",
                                    num_cores=_NC, num_subcores=_NS),
        compiler_params=pltpu.CompilerParams(use_tc_tiling_on_sc=False),
        scratch_types=(
            [pltpu.VMEM((rpw,), jnp.int32)] * 2
            + [pltpu.VMEM((_CHUNK, _D), _F32)] * 4
            + [pltpu.SemaphoreType.DMA] * 4
        ),
    )
    uc_g, ic_g = sc_gather(uidx, iidx, ucat, icat)

    H1 = W1.shape[0]
    H2 = W2.shape[0]
    tile = _TILE
    nt = B // tile
    w1t = W1.T
    w2t = W2.T

    def full(a):
        return pl.BlockSpec(a.shape, lambda i: (0,) * a.ndim)

    row_spec = lambda w: pl.BlockSpec((tile, w), lambda i: (i, 0))
    st_spec = lambda w: pl.BlockSpec((2, w), lambda i: (0, 0))

    b1r, g1r, be1r = (v.reshape(1, -1) for v in (b1, g1, be1))
    b2r, g2r, be2r = (v.reshape(1, -1) for v in (b2, g2, be2))

    w1ut_b = w1t[:_D].astype(jnp.bfloat16)
    w1it_b = w1t[_D:].astype(jnp.bfloat16)
    y1, st1, xg = pl.pallas_call(
        _stage1_body,
        grid=(nt,),
        in_specs=[row_spec(_D), row_spec(_D),
                  full(w1ut_b), full(w1it_b), full(b1r)],
        out_specs=[row_spec(H1), st_spec(H1), row_spec(_D)],
        out_shape=[jax.ShapeDtypeStruct((B, H1), _F32),
                   jax.ShapeDtypeStruct((2, H1), _F32),
                   jax.ShapeDtypeStruct((B, _D), _F32)],
    )(uc_g, ic_g, w1ut_b, w1it_b, b1r)

    y2, st2 = pl.pallas_call(
        functools.partial(_stage2_body, n=float(B)),
        grid=(nt,),
        in_specs=[row_spec(H1), st_spec(H1),
                  full(g1r), full(be1r), full(w2t), full(b2r)],
        out_specs=[row_spec(H2), st_spec(H2)],
        out_shape=[jax.ShapeDtypeStruct((B, H2), _F32),
                   jax.ShapeDtypeStruct((2, H2), _F32)],
    )(y1, st1, g1r, be1r, w2t, b2r)

    wg = Wout[:, :_D]
    wh = Wout[:, _D:]
    bor = bout.reshape(1, 1)
    out2 = pl.pallas_call(
        functools.partial(_stage3_body, n=float(B)),
        grid=(nt,),
        in_specs=[row_spec(H2), st_spec(H2), full(g2r), full(be2r),
                  row_spec(_D), full(wg), full(wh), full(bor)],
        out_specs=pl.BlockSpec((tile, 1), lambda i: (i, 0)),
        out_shape=jax.ShapeDtypeStruct((B, 1), _F32),
    )(y2, st2, g2r, be2r, xg, wg, wh, bor)
    return out2.reshape(B)


# speedup vs baseline: 1.9742x; 1.9742x over previous
import functools

import jax
import jax.numpy as jnp
from jax import lax
from jax.experimental import pallas as pl
from jax.experimental.pallas import tpu as pltpu
from jax.experimental.pallas import tpu_sc as plsc

_D = 64
_NC, _NS = 2, 16
_NW = _NC * _NS
_CHUNK = 128
_TBLK = 2048
_TILE = 2048
_EPS = 1e-5
_F32 = jnp.float32
_HI = lax.Precision.HIGHEST


def _transcat_body(gu, mu, gi, mi, out_ref):
    def pack(g, m):
        lo = lax.bitcast_convert_type(g[...].astype(jnp.bfloat16),
                                      jnp.uint16).astype(jnp.uint32)
        hi = lax.bitcast_convert_type(m[...].astype(jnp.bfloat16),
                                      jnp.uint16).astype(jnp.uint32)
        return lax.bitcast_convert_type(lo | (hi << 16), _F32)

    tu = jnp.transpose(pack(gu, mu), (1, 0))
    ti = jnp.transpose(pack(gi, mi), (1, 0))
    out_ref[...] = jnp.concatenate([tu, ti], axis=1)


def _transcat(gut, mut, git, mit):
    V = gut.shape[1]
    n = -(-V // _TBLK)
    return pl.pallas_call(
        _transcat_body,
        grid=(n,),
        in_specs=[pl.BlockSpec((_D, _TBLK), lambda i: (0, i))] * 4,
        out_specs=pl.BlockSpec((_TBLK, 2 * _D), lambda i: (i, 0)),
        out_shape=jax.ShapeDtypeStruct((V, 2 * _D), _F32),
    )(gut, mut, git, mit)


def _sc_gather_body(uidx, iidx, cat, out_u, out_i,
                    idxu_v, idxi_v, b0, b1, b2, b3, s0, s1, s2, s3):
    rpw = idxu_v.shape[0]
    cpt = rpw // _CHUNK
    wid = lax.axis_index("s") * _NC + lax.axis_index("c")
    base = wid * rpw
    pltpu.sync_copy(uidx.at[pl.ds(base, rpw)], idxu_v)
    pltpu.sync_copy(iidx.at[pl.ds(base, rpw)], idxi_v)

    bufs = (b0, b1, b2, b3)
    sems = (s0, s1, s2, s3)
    units = ([(idxu_v, out_u, 0, j) for j in range(cpt)]
             + [(idxi_v, out_i, _D, j) for j in range(cpt)])
    nbuf = len(bufs)
    copies = {}

    def fire(t):
        idxv, _, _, j = units[t]
        copies[t] = pltpu.async_copy(
            cat.at[idxv.at[pl.ds(j * _CHUNK, _CHUNK)]],
            bufs[t % nbuf], sems[t % nbuf])

    def drain(t):
        copies[t].wait()
        _, out, col, j = units[t]
        pltpu.sync_copy(bufs[t % nbuf].at[pl.ds(0, _CHUNK), pl.ds(col, _D)],
                        out.at[pl.ds(base + j * _CHUNK, _CHUNK)])

    for t in range(len(units)):
        if t >= nbuf:
            drain(t - nbuf)
        fire(t)
    for t in range(len(units) - nbuf, len(units)):
        drain(t)


def _accum_stats(i, y, st_ref):
    ps = jnp.sum(y, axis=0, keepdims=True)
    pq = jnp.sum(y * y, axis=0, keepdims=True)
    part = jnp.concatenate([ps, pq], axis=0)

    @pl.when(i == 0)
    def _():
        st_ref[...] = part

    @pl.when(i > 0)
    def _():
        st_ref[...] += part


def _bn_relu_from_stats(st_ref, n, g, be, y):
    mean = st_ref[0:1, :] * (1.0 / n)
    var = st_ref[1:2, :] * (1.0 / n) - mean * mean
    return jnp.maximum((y - mean) * lax.rsqrt(var + _EPS) * g + be, 0.0)


def _unpack_pair(w):
    u = lax.bitcast_convert_type(w, jnp.uint32)
    g = lax.bitcast_convert_type((u & 0xFFFF).astype(jnp.uint16), jnp.bfloat16)
    m = lax.bitcast_convert_type((u >> 16).astype(jnp.uint16), jnp.bfloat16)
    return g, m


def _stage1_body(uc, ic, w1ut, w1it, b1, y1_ref, st1_ref, xg_ref):
    i = pl.program_id(0)
    ug, um = _unpack_pair(uc[...])
    ig, im = _unpack_pair(ic[...])
    y1 = (jnp.dot(um, w1ut[...], preferred_element_type=_F32)
          + jnp.dot(im, w1it[...], preferred_element_type=_F32)
          + b1[...])
    y1_ref[...] = y1
    xg_ref[...] = ug.astype(_F32) * ig.astype(_F32)
    _accum_stats(i, y1, st1_ref)


def _stage2_body(y1, st1, g1, be1, w2t, b2, y2_ref, st2_ref, *, n):
    i = pl.program_id(0)
    h1 = _bn_relu_from_stats(st1, n, g1[...], be1[...], y1[...])
    y2 = jnp.dot(h1, w2t[...], preferred_element_type=_F32, precision=_HI) + b2[...]
    y2_ref[...] = y2
    _accum_stats(i, y2, st2_ref)


def _stage3_body(y2, st2, g2, be2, xg, wg, wh, bo, out_ref, *, n):
    h2 = _bn_relu_from_stats(st2, n, g2[...], be2[...], y2[...])
    s = (jnp.sum(xg[...] * wg[...], axis=1, keepdims=True)
         + jnp.sum(h2 * wh[...], axis=1, keepdims=True) + bo[...])
    out_ref[...] = 1.0 / (1.0 + jnp.exp(-s))


def kernel(user_idx, item_idx, gmf_user, gmf_item, mlp_user, mlp_item,
           W1, b1, g1, be1, W2, b2, g2, be2, Wout, bout):
    B = user_idx.shape[0]
    uidx = user_idx.astype(jnp.int32)
    iidx = item_idx.astype(jnp.int32)

    cat = _transcat(gmf_user.T, mlp_user.T, gmf_item.T, mlp_item.T)

    rpw = B // _NW
    sc_gather = pl.kernel(
        _sc_gather_body,
        out_type=(jax.ShapeDtypeStruct((B, _D), _F32),) * 2,
        mesh=plsc.VectorSubcoreMesh(core_axis_name="c", subcore_axis_name="s",
                                    num_cores=_NC, num_subcores=_NS),
        compiler_params=pltpu.CompilerParams(use_tc_tiling_on_sc=False),
        scratch_types=(
            [pltpu.VMEM((rpw,), jnp.int32)] * 2
            + [pltpu.VMEM((_CHUNK, 2 * _D), _F32)] * 4
            + [pltpu.SemaphoreType.DMA] * 4
        ),
    )
    uc_g, ic_g = sc_gather(uidx, iidx, cat)

    H1 = W1.shape[0]
    H2 = W2.shape[0]
    tile = _TILE
    nt = B // tile
    w1t = W1.T
    w2t = W2.T

    def full(a):
        return pl.BlockSpec(a.shape, lambda i: (0,) * a.ndim)

    row_spec = lambda w: pl.BlockSpec((tile, w), lambda i: (i, 0))
    st_spec = lambda w: pl.BlockSpec((2, w), lambda i: (0, 0))

    b1r, g1r, be1r = (v.reshape(1, -1) for v in (b1, g1, be1))
    b2r, g2r, be2r = (v.reshape(1, -1) for v in (b2, g2, be2))

    w1ut_b = w1t[:_D].astype(jnp.bfloat16)
    w1it_b = w1t[_D:].astype(jnp.bfloat16)
    y1, st1, xg = pl.pallas_call(
        _stage1_body,
        grid=(nt,),
        in_specs=[row_spec(_D), row_spec(_D),
                  full(w1ut_b), full(w1it_b), full(b1r)],
        out_specs=[row_spec(H1), st_spec(H1), row_spec(_D)],
        out_shape=[jax.ShapeDtypeStruct((B, H1), _F32),
                   jax.ShapeDtypeStruct((2, H1), _F32),
                   jax.ShapeDtypeStruct((B, _D), _F32)],
    )(uc_g, ic_g, w1ut_b, w1it_b, b1r)

    y2, st2 = pl.pallas_call(
        functools.partial(_stage2_body, n=float(B)),
        grid=(nt,),
        in_specs=[row_spec(H1), st_spec(H1),
                  full(g1r), full(be1r), full(w2t), full(b2r)],
        out_specs=[row_spec(H2), st_spec(H2)],
        out_shape=[jax.ShapeDtypeStruct((B, H2), _F32),
                   jax.ShapeDtypeStruct((2, H2), _F32)],
    )(y1, st1, g1r, be1r, w2t, b2r)

    wg = Wout[:, :_D]
    wh = Wout[:, _D:]
    bor = bout.reshape(1, 1)
    out2 = pl.pallas_call(
        functools.partial(_stage3_body, n=float(B)),
        grid=(nt,),
        in_specs=[row_spec(H2), st_spec(H2), full(g2r), full(be2r),
                  row_spec(_D), full(wg), full(wh), full(bor)],
        out_specs=pl.BlockSpec((tile, 1), lambda i: (i, 0)),
        out_shape=jax.ShapeDtypeStruct((B, 1), _F32),
    )(y2, st2, g2r, be2r, xg, wg, wh, bor)
    return out2.reshape(B)


# speedup vs baseline: 2.2804x; 1.1551x over previous
import functools

import jax
import jax.numpy as jnp
from jax import lax
from jax.experimental import pallas as pl
from jax.experimental.pallas import tpu as pltpu
from jax.experimental.pallas import tpu_sc as plsc

_D = 64
_NC, _NS = 2, 16
_NW = _NC * _NS
_CHUNK = 128
_TBLK = 4096
_TILE = 2048
_EPS = 1e-5
_F32 = jnp.float32
_HI = lax.Precision.HIGHEST


def _transcat_body(gu, mu, gi, mi, out_ref):
    def pack(g, m):
        lo = lax.bitcast_convert_type(g[...].astype(jnp.bfloat16),
                                      jnp.uint16).astype(jnp.uint32)
        hi = lax.bitcast_convert_type(m[...].astype(jnp.bfloat16),
                                      jnp.uint16).astype(jnp.uint32)
        return lax.bitcast_convert_type(lo | (hi << 16), _F32)

    tu = jnp.transpose(pack(gu, mu), (1, 0))
    ti = jnp.transpose(pack(gi, mi), (1, 0))
    out_ref[...] = jnp.concatenate([tu, ti], axis=1)


def _transcat(gut, mut, git, mit):
    V = gut.shape[1]
    n = -(-V // _TBLK)
    return pl.pallas_call(
        _transcat_body,
        grid=(n,),
        in_specs=[pl.BlockSpec((_D, _TBLK), lambda i: (0, i))] * 4,
        out_specs=pl.BlockSpec((_TBLK, 2 * _D), lambda i: (i, 0)),
        out_shape=jax.ShapeDtypeStruct((V, 2 * _D), _F32),
    )(gut, mut, git, mit)


def _sc_gather_body(uidx, iidx, cat, out_u, out_i,
                    idxu_v, idxi_v, b0, b1, b2, b3, s0, s1, s2, s3):
    rpw = idxu_v.shape[0]
    cpt = rpw // _CHUNK
    wid = lax.axis_index("s") * _NC + lax.axis_index("c")
    base = wid * rpw
    pltpu.sync_copy(uidx.at[pl.ds(base, rpw)], idxu_v)
    pltpu.sync_copy(iidx.at[pl.ds(base, rpw)], idxi_v)

    bufs = (b0, b1, b2, b3)
    sems = (s0, s1, s2, s3)
    units = ([(idxu_v, out_u, 0, j) for j in range(cpt)]
             + [(idxi_v, out_i, _D, j) for j in range(cpt)])
    nbuf = len(bufs)
    copies = {}

    def fire(t):
        idxv, _, _, j = units[t]
        copies[t] = pltpu.async_copy(
            cat.at[idxv.at[pl.ds(j * _CHUNK, _CHUNK)]],
            bufs[t % nbuf], sems[t % nbuf])

    def drain(t):
        copies[t].wait()
        _, out, col, j = units[t]
        pltpu.sync_copy(bufs[t % nbuf].at[pl.ds(0, _CHUNK), pl.ds(col, _D)],
                        out.at[pl.ds(base + j * _CHUNK, _CHUNK)])

    for t in range(len(units)):
        if t >= nbuf:
            drain(t - nbuf)
        fire(t)
    for t in range(len(units) - nbuf, len(units)):
        drain(t)


def _accum_stats(i, y, st_ref):
    ps = jnp.sum(y, axis=0, keepdims=True)
    pq = jnp.sum(y * y, axis=0, keepdims=True)
    part = jnp.concatenate([ps, pq], axis=0)

    @pl.when(i == 0)
    def _():
        st_ref[...] = part

    @pl.when(i > 0)
    def _():
        st_ref[...] += part


def _bn_relu_from_stats(st_ref, n, g, be, y):
    mean = st_ref[0:1, :] * (1.0 / n)
    var = st_ref[1:2, :] * (1.0 / n) - mean * mean
    return jnp.maximum((y - mean) * lax.rsqrt(var + _EPS) * g + be, 0.0)


def _unpack_pair(w):
    u = lax.bitcast_convert_type(w, jnp.uint32)
    g = lax.bitcast_convert_type((u & 0xFFFF).astype(jnp.uint16), jnp.bfloat16)
    m = lax.bitcast_convert_type((u >> 16).astype(jnp.uint16), jnp.bfloat16)
    return g, m


def _stage1_body(uc, ic, w1ut, w1it, b1, y1_ref, st1_ref, xg_ref):
    i = pl.program_id(0)
    ug, um = _unpack_pair(uc[...])
    ig, im = _unpack_pair(ic[...])
    y1 = (jnp.dot(um, w1ut[...], preferred_element_type=_F32)
          + jnp.dot(im, w1it[...], preferred_element_type=_F32)
          + b1[...])
    y1_ref[...] = y1.astype(jnp.bfloat16)
    xg_ref[...] = (ug.astype(_F32) * ig.astype(_F32)).astype(jnp.bfloat16)
    _accum_stats(i, y1, st1_ref)


def _stage2_body(y1, st1, g1, be1, w2t, b2, y2_ref, st2_ref, *, n):
    i = pl.program_id(0)
    h1 = _bn_relu_from_stats(st1, n, g1[...], be1[...], y1[...].astype(_F32))
    y2 = jnp.dot(h1.astype(jnp.bfloat16), w2t[...],
                 preferred_element_type=_F32) + b2[...]
    y2_ref[...] = y2.astype(jnp.bfloat16)
    _accum_stats(i, y2, st2_ref)


def _stage3_body(y2, st2, g2, be2, xg, wg, wh, bo, out_ref, *, n):
    h2 = _bn_relu_from_stats(st2, n, g2[...], be2[...], y2[...].astype(_F32))
    s = (jnp.sum(xg[...].astype(_F32) * wg[...], axis=1, keepdims=True)
         + jnp.sum(h2 * wh[...], axis=1, keepdims=True) + bo[...])
    out_ref[...] = 1.0 / (1.0 + jnp.exp(-s))


def kernel(user_idx, item_idx, gmf_user, gmf_item, mlp_user, mlp_item,
           W1, b1, g1, be1, W2, b2, g2, be2, Wout, bout):
    B = user_idx.shape[0]
    uidx = user_idx.astype(jnp.int32)
    iidx = item_idx.astype(jnp.int32)

    cat = _transcat(gmf_user.T, mlp_user.T, gmf_item.T, mlp_item.T)

    rpw = B // _NW
    sc_gather = pl.kernel(
        _sc_gather_body,
        out_type=(jax.ShapeDtypeStruct((B, _D), _F32),) * 2,
        mesh=plsc.VectorSubcoreMesh(core_axis_name="c", subcore_axis_name="s",
                                    num_cores=_NC, num_subcores=_NS),
        compiler_params=pltpu.CompilerParams(use_tc_tiling_on_sc=False),
        scratch_types=(
            [pltpu.VMEM((rpw,), jnp.int32)] * 2
            + [pltpu.VMEM((_CHUNK, 2 * _D), _F32)] * 4
            + [pltpu.SemaphoreType.DMA] * 4
        ),
    )
    uc_g, ic_g = sc_gather(uidx, iidx, cat)

    H1 = W1.shape[0]
    H2 = W2.shape[0]
    tile = _TILE
    nt = B // tile
    w1t = W1.T
    w2t = W2.T

    def full(a):
        return pl.BlockSpec(a.shape, lambda i: (0,) * a.ndim)

    row_spec = lambda w: pl.BlockSpec((tile, w), lambda i: (i, 0))
    st_spec = lambda w: pl.BlockSpec((2, w), lambda i: (0, 0))

    b1r, g1r, be1r = (v.reshape(1, -1) for v in (b1, g1, be1))
    b2r, g2r, be2r = (v.reshape(1, -1) for v in (b2, g2, be2))

    w1ut_b = w1t[:_D].astype(jnp.bfloat16)
    w1it_b = w1t[_D:].astype(jnp.bfloat16)
    y1, st1, xg = pl.pallas_call(
        _stage1_body,
        grid=(nt,),
        in_specs=[row_spec(_D), row_spec(_D),
                  full(w1ut_b), full(w1it_b), full(b1r)],
        out_specs=[row_spec(H1), st_spec(H1), row_spec(_D)],
        out_shape=[jax.ShapeDtypeStruct((B, H1), jnp.bfloat16),
                   jax.ShapeDtypeStruct((2, H1), _F32),
                   jax.ShapeDtypeStruct((B, _D), jnp.bfloat16)],
    )(uc_g, ic_g, w1ut_b, w1it_b, b1r)

    y2, st2 = pl.pallas_call(
        functools.partial(_stage2_body, n=float(B)),
        grid=(nt,),
        in_specs=[row_spec(H1), st_spec(H1),
                  full(g1r), full(be1r), full(w2t), full(b2r)],
        out_specs=[row_spec(H2), st_spec(H2)],
        out_shape=[jax.ShapeDtypeStruct((B, H2), jnp.bfloat16),
                   jax.ShapeDtypeStruct((2, H2), _F32)],
    )(y1, st1, g1r, be1r, w2t.astype(jnp.bfloat16), b2r)

    wg = Wout[:, :_D]
    wh = Wout[:, _D:]
    bor = bout.reshape(1, 1)
    out2 = pl.pallas_call(
        functools.partial(_stage3_body, n=float(B)),
        grid=(nt,),
        in_specs=[row_spec(H2), st_spec(H2), full(g2r), full(be2r),
                  row_spec(_D), full(wg), full(wh), full(bor)],
        out_specs=pl.BlockSpec((tile, 1), lambda i: (i, 0)),
        out_shape=jax.ShapeDtypeStruct((B, 1), _F32),
    )(y2, st2, g2r, be2r, xg, wg, wh, bor)
    return out2.reshape(B)


# speedup vs baseline: 2.4924x; 1.0930x over previous
import functools

import jax
import jax.numpy as jnp
from jax import lax
from jax.experimental import pallas as pl
from jax.experimental.pallas import tpu as pltpu
from jax.experimental.pallas import tpu_sc as plsc

_D = 64
_NC, _NS = 2, 16
_NW = _NC * _NS
_CHUNK = 128
_TBLK = 4096
_TILE = 2048
_EPS = 1e-5
_F32 = jnp.float32
_HI = lax.Precision.HIGHEST


def _transcat_body(gu, mu, gi, mi, out_ref):
    def pack(g, m):
        lo = lax.bitcast_convert_type(g[...].astype(jnp.bfloat16),
                                      jnp.uint16).astype(jnp.uint32)
        hi = lax.bitcast_convert_type(m[...].astype(jnp.bfloat16),
                                      jnp.uint16).astype(jnp.uint32)
        return lax.bitcast_convert_type(lo | (hi << 16), _F32)

    tu = jnp.transpose(pack(gu, mu), (1, 0))
    ti = jnp.transpose(pack(gi, mi), (1, 0))
    out_ref[...] = jnp.concatenate([tu, ti], axis=1)


def _transcat(gut, mut, git, mit):
    V = gut.shape[1]
    n = -(-V // _TBLK)
    return pl.pallas_call(
        _transcat_body,
        grid=(n,),
        in_specs=[pl.BlockSpec((_D, _TBLK), lambda i: (0, i))] * 4,
        out_specs=pl.BlockSpec((_TBLK, 2 * _D), lambda i: (i, 0)),
        out_shape=jax.ShapeDtypeStruct((V, 2 * _D), _F32),
    )(gut, mut, git, mit)


def _sc_gather_body(uidx, iidx, cat, out_u, out_i,
                    idxu_v, idxi_v, b0, b1, b2, b3, s0, s1, s2, s3):
    rpw = idxu_v.shape[0]
    cpt = rpw // _CHUNK
    wid = lax.axis_index("s") * _NC + lax.axis_index("c")
    base = wid * rpw
    pltpu.sync_copy(uidx.at[pl.ds(base, rpw)], idxu_v)
    pltpu.sync_copy(iidx.at[pl.ds(base, rpw)], idxi_v)

    bufs = (b0, b1, b2, b3)
    sems = (s0, s1, s2, s3)
    units = ([(idxu_v, out_u, j) for j in range(cpt)]
             + [(idxi_v, out_i, j) for j in range(cpt)])
    nbuf = len(bufs)
    copies = {}

    def fire(t):
        idxv, _, j = units[t]
        copies[t] = pltpu.async_copy(
            cat.at[idxv.at[pl.ds(j * _CHUNK, _CHUNK)]],
            bufs[t % nbuf], sems[t % nbuf])

    def drain(t):
        copies[t].wait()
        _, out, j = units[t]
        pltpu.sync_copy(bufs[t % nbuf],
                        out.at[pl.ds(base + j * _CHUNK, _CHUNK)])

    for t in range(len(units)):
        if t >= nbuf:
            drain(t - nbuf)
        fire(t)
    for t in range(len(units) - nbuf, len(units)):
        drain(t)


def _accum_stats(i, y, st_ref):
    ps = jnp.sum(y, axis=0, keepdims=True)
    pq = jnp.sum(y * y, axis=0, keepdims=True)
    part = jnp.concatenate([ps, pq], axis=0)

    @pl.when(i == 0)
    def _():
        st_ref[...] = part

    @pl.when(i > 0)
    def _():
        st_ref[...] += part


def _bn_relu_from_stats(st_ref, n, g, be, y):
    mean = st_ref[0:1, :] * (1.0 / n)
    var = st_ref[1:2, :] * (1.0 / n) - mean * mean
    return jnp.maximum((y - mean) * lax.rsqrt(var + _EPS) * g + be, 0.0)


def _unpack_pair(w):
    u = lax.bitcast_convert_type(w, jnp.uint32)
    g = lax.bitcast_convert_type((u & 0xFFFF).astype(jnp.uint16), jnp.bfloat16)
    m = lax.bitcast_convert_type((u >> 16).astype(jnp.uint16), jnp.bfloat16)
    return g, m


def _stage1_body(uc, ic, w1ut, w1it, b1, y1_ref, st1_ref, xg_ref):
    i = pl.program_id(0)
    ug, um = _unpack_pair(uc[...][:, :_D])
    ig, im = _unpack_pair(ic[...][:, _D:])
    y1 = (jnp.dot(um, w1ut[...], preferred_element_type=_F32)
          + jnp.dot(im, w1it[...], preferred_element_type=_F32)
          + b1[...])
    y1_ref[...] = y1.astype(jnp.bfloat16)
    xg_ref[...] = (ug.astype(_F32) * ig.astype(_F32)).astype(jnp.bfloat16)
    _accum_stats(i, y1, st1_ref)


def _stage2_body(y1, st1, g1, be1, w2t, b2, y2_ref, st2_ref, *, n):
    i = pl.program_id(0)
    h1 = _bn_relu_from_stats(st1, n, g1[...], be1[...], y1[...].astype(_F32))
    y2 = jnp.dot(h1.astype(jnp.bfloat16), w2t[...],
                 preferred_element_type=_F32) + b2[...]
    y2_ref[...] = y2.astype(jnp.bfloat16)
    _accum_stats(i, y2, st2_ref)


def _stage3_body(y2, st2, g2, be2, xg, wg, wh, bo, out_ref, *, n):
    h2 = _bn_relu_from_stats(st2, n, g2[...], be2[...], y2[...].astype(_F32))
    s = (jnp.sum(xg[...].astype(_F32) * wg[...], axis=1)
         + jnp.sum(h2 * wh[...], axis=1) + bo[0, 0])
    out_ref[...] = 1.0 / (1.0 + jnp.exp(-s))


def kernel(user_idx, item_idx, gmf_user, gmf_item, mlp_user, mlp_item,
           W1, b1, g1, be1, W2, b2, g2, be2, Wout, bout):
    B = user_idx.shape[0]
    uidx = user_idx.astype(jnp.int32)
    iidx = item_idx.astype(jnp.int32)

    cat = _transcat(gmf_user.T, mlp_user.T, gmf_item.T, mlp_item.T)

    rpw = B // _NW
    sc_gather = pl.kernel(
        _sc_gather_body,
        out_type=(jax.ShapeDtypeStruct((B, 2 * _D), _F32),) * 2,
        mesh=plsc.VectorSubcoreMesh(core_axis_name="c", subcore_axis_name="s",
                                    num_cores=_NC, num_subcores=_NS),
        compiler_params=pltpu.CompilerParams(use_tc_tiling_on_sc=False),
        scratch_types=(
            [pltpu.VMEM((rpw,), jnp.int32)] * 2
            + [pltpu.VMEM((_CHUNK, 2 * _D), _F32)] * 4
            + [pltpu.SemaphoreType.DMA] * 4
        ),
    )
    uc_g, ic_g = sc_gather(uidx, iidx, cat)

    H1 = W1.shape[0]
    H2 = W2.shape[0]
    tile = _TILE
    nt = B // tile
    w1t = W1.T
    w2t = W2.T

    def full(a):
        return pl.BlockSpec(a.shape, lambda i: (0,) * a.ndim)

    row_spec = lambda w: pl.BlockSpec((tile, w), lambda i: (i, 0))
    st_spec = lambda w: pl.BlockSpec((2, w), lambda i: (0, 0))

    b1r, g1r, be1r = (v.reshape(1, -1) for v in (b1, g1, be1))
    b2r, g2r, be2r = (v.reshape(1, -1) for v in (b2, g2, be2))

    w1ut_b = w1t[:_D].astype(jnp.bfloat16)
    w1it_b = w1t[_D:].astype(jnp.bfloat16)
    y1, st1, xg = pl.pallas_call(
        _stage1_body,
        grid=(nt,),
        in_specs=[row_spec(2 * _D), row_spec(2 * _D),
                  full(w1ut_b), full(w1it_b), full(b1r)],
        out_specs=[row_spec(H1), st_spec(H1), row_spec(_D)],
        out_shape=[jax.ShapeDtypeStruct((B, H1), jnp.bfloat16),
                   jax.ShapeDtypeStruct((2, H1), _F32),
                   jax.ShapeDtypeStruct((B, _D), jnp.bfloat16)],
    )(uc_g, ic_g, w1ut_b, w1it_b, b1r)

    y2, st2 = pl.pallas_call(
        functools.partial(_stage2_body, n=float(B)),
        grid=(nt,),
        in_specs=[row_spec(H1), st_spec(H1),
                  full(g1r), full(be1r), full(w2t), full(b2r)],
        out_specs=[row_spec(H2), st_spec(H2)],
        out_shape=[jax.ShapeDtypeStruct((B, H2), jnp.bfloat16),
                   jax.ShapeDtypeStruct((2, H2), _F32)],
    )(y1, st1, g1r, be1r, w2t.astype(jnp.bfloat16), b2r)

    wg = Wout[:, :_D]
    wh = Wout[:, _D:]
    bor = bout.reshape(1, 1)
    out1 = pl.pallas_call(
        functools.partial(_stage3_body, n=float(B)),
        grid=(nt,),
        in_specs=[row_spec(H2), st_spec(H2), full(g2r), full(be2r),
                  row_spec(_D), full(wg), full(wh), full(bor)],
        out_specs=pl.BlockSpec((tile,), lambda i: (i,)),
        out_shape=jax.ShapeDtypeStruct((B,), _F32),
    )(y2, st2, g2r, be2r, xg, wg, wh, bor)
    return out1


# speedup vs baseline: 2.6275x; 1.0542x over previous
import functools

import jax
import jax.numpy as jnp
from jax import lax
from jax.experimental import pallas as pl
from jax.experimental.pallas import tpu as pltpu
from jax.experimental.pallas import tpu_sc as plsc

_D = 64
_NC, _NS = 2, 16
_NW = _NC * _NS
_CHUNK = 128
_TBLK = 4096
_TILE = 2048
_EPS = 1e-5
_F32 = jnp.float32
_HI = lax.Precision.HIGHEST


def _transcat_body(gu, mu, gi, mi, out_ref):
    def pack(g, m):
        lo = lax.bitcast_convert_type(g[...].astype(jnp.bfloat16),
                                      jnp.uint16).astype(jnp.uint32)
        hi = lax.bitcast_convert_type(m[...].astype(jnp.bfloat16),
                                      jnp.uint16).astype(jnp.uint32)
        return lax.bitcast_convert_type(lo | (hi << 16), _F32)

    tu = jnp.transpose(pack(gu, mu), (1, 0))
    ti = jnp.transpose(pack(gi, mi), (1, 0))
    out_ref[...] = jnp.concatenate([tu, ti], axis=1)


def _transcat(gut, mut, git, mit):
    V = gut.shape[1]
    n = -(-V // _TBLK)
    return pl.pallas_call(
        _transcat_body,
        grid=(n,),
        in_specs=[pl.BlockSpec((_D, _TBLK), lambda i: (0, i))] * 4,
        out_specs=pl.BlockSpec((_TBLK, 2 * _D), lambda i: (i, 0)),
        out_shape=jax.ShapeDtypeStruct((V, 2 * _D), _F32),
    )(gut, mut, git, mit)


def _sc_gather_body(uidx, iidx, cat, out_u, out_i,
                    idxu_v, idxi_v, b0, b1, b2, b3, s0, s1, s2, s3):
    rpw = idxu_v.shape[0]
    cpt = rpw // _CHUNK
    wid = lax.axis_index("s") * _NC + lax.axis_index("c")
    base = wid * rpw
    pltpu.sync_copy(uidx.at[pl.ds(base, rpw)], idxu_v)
    pltpu.sync_copy(iidx.at[pl.ds(base, rpw)], idxi_v)

    bufs = (b0, b1, b2, b3)
    sems = (s0, s1, s2, s3)
    units = ([(idxu_v, out_u, j) for j in range(cpt)]
             + [(idxi_v, out_i, j) for j in range(cpt)])
    nbuf = len(bufs)
    copies = {}

    def fire(t):
        idxv, _, j = units[t]
        copies[t] = pltpu.async_copy(
            cat.at[idxv.at[pl.ds(j * _CHUNK, _CHUNK)]],
            bufs[t % nbuf], sems[t % nbuf])

    def drain(t):
        copies[t].wait()
        _, out, j = units[t]
        pltpu.sync_copy(bufs[t % nbuf],
                        out.at[pl.ds(base + j * _CHUNK, _CHUNK)])

    for t in range(len(units)):
        if t >= nbuf:
            drain(t - nbuf)
        fire(t)
    for t in range(len(units) - nbuf, len(units)):
        drain(t)


def _accum_stats(i, y, st_ref):
    ps = jnp.sum(y, axis=0, keepdims=True)
    pq = jnp.sum(y * y, axis=0, keepdims=True)
    part = jnp.concatenate([ps, pq], axis=0)

    @pl.when(i == 0)
    def _():
        st_ref[...] = part

    @pl.when(i > 0)
    def _():
        st_ref[...] += part


def _bn_relu_from_stats(st_ref, n, g, be, y):
    mean = st_ref[0:1, :] * (1.0 / n)
    var = st_ref[1:2, :] * (1.0 / n) - mean * mean
    return jnp.maximum((y - mean) * lax.rsqrt(var + _EPS) * g + be, 0.0)


def _unpack_pair(w):
    u = lax.bitcast_convert_type(w, jnp.uint32)
    g = lax.bitcast_convert_type((u & 0xFFFF).astype(jnp.uint16), jnp.bfloat16)
    m = lax.bitcast_convert_type((u >> 16).astype(jnp.uint16), jnp.bfloat16)
    return g, m


def _dense_body(uc, ic, w1ut, w1it, b1, g1, be1, w2t, b2, g2, be2,
                wg, wh, bo, out_ref, y1_s, xg_s, st1_s, y2_s, st2_s,
                *, n, tile):
    p = pl.program_id(0)
    i = pl.program_id(1)
    rows = pl.ds(i * tile, tile)
    bf16 = jnp.bfloat16

    @pl.when(p == 0)
    def _():
        ug, um = _unpack_pair(uc[...][:, :_D])
        ig, im = _unpack_pair(ic[...][:, _D:])
        y1 = (jnp.dot(um, w1ut[...], preferred_element_type=_F32)
              + jnp.dot(im, w1it[...], preferred_element_type=_F32)
              + b1[...])
        y1_s[rows, :] = y1.astype(bf16)
        xg_s[rows, :] = (ug.astype(_F32) * ig.astype(_F32)).astype(bf16)
        _accum_stats(i, y1, st1_s)

    @pl.when(p == 1)
    def _():
        h1 = _bn_relu_from_stats(st1_s, n, g1[...], be1[...],
                                 y1_s[rows, :].astype(_F32))
        y2 = jnp.dot(h1.astype(bf16), w2t[...],
                     preferred_element_type=_F32) + b2[...]
        y2_s[rows, :] = y2.astype(bf16)
        _accum_stats(i, y2, st2_s)

    @pl.when(p == 2)
    def _():
        h2 = _bn_relu_from_stats(st2_s, n, g2[...], be2[...],
                                 y2_s[rows, :].astype(_F32))
        s = (jnp.sum(xg_s[rows, :].astype(_F32) * wg[...], axis=1)
             + jnp.sum(h2 * wh[...], axis=1) + bo[0, 0])
        out_ref[...] = 1.0 / (1.0 + jnp.exp(-s))


def kernel(user_idx, item_idx, gmf_user, gmf_item, mlp_user, mlp_item,
           W1, b1, g1, be1, W2, b2, g2, be2, Wout, bout):
    B = user_idx.shape[0]
    uidx = user_idx.astype(jnp.int32)
    iidx = item_idx.astype(jnp.int32)

    cat = _transcat(gmf_user.T, mlp_user.T, gmf_item.T, mlp_item.T)

    rpw = B // _NW
    sc_gather = pl.kernel(
        _sc_gather_body,
        out_type=(jax.ShapeDtypeStruct((B, 2 * _D), _F32),) * 2,
        mesh=plsc.VectorSubcoreMesh(core_axis_name="c", subcore_axis_name="s",
                                    num_cores=_NC, num_subcores=_NS),
        compiler_params=pltpu.CompilerParams(use_tc_tiling_on_sc=False),
        scratch_types=(
            [pltpu.VMEM((rpw,), jnp.int32)] * 2
            + [pltpu.VMEM((_CHUNK, 2 * _D), _F32)] * 4
            + [pltpu.SemaphoreType.DMA] * 4
        ),
    )
    uc_g, ic_g = sc_gather(uidx, iidx, cat)

    H1 = W1.shape[0]
    H2 = W2.shape[0]
    tile = _TILE
    nt = B // tile
    w1t = W1.T
    w2t = W2.T

    def full(a):
        return pl.BlockSpec(a.shape, lambda p, i: (0,) * a.ndim)

    gath_spec = pl.BlockSpec((tile, 2 * _D), lambda p, i: (i * (p == 0), 0))

    b1r, g1r, be1r = (v.reshape(1, -1) for v in (b1, g1, be1))
    b2r, g2r, be2r = (v.reshape(1, -1) for v in (b2, g2, be2))

    w1ut_b = w1t[:_D].astype(jnp.bfloat16)
    w1it_b = w1t[_D:].astype(jnp.bfloat16)
    w2t_b = w2t.astype(jnp.bfloat16)
    wg = Wout[:, :_D]
    wh = Wout[:, _D:]
    bor = bout.reshape(1, 1)
    out1 = pl.pallas_call(
        functools.partial(_dense_body, n=float(B), tile=tile),
        grid=(3, nt),
        in_specs=[gath_spec, gath_spec,
                  full(w1ut_b), full(w1it_b), full(b1r),
                  full(g1r), full(be1r), full(w2t_b), full(b2r),
                  full(g2r), full(be2r), full(wg), full(wh), full(bor)],
        out_specs=pl.BlockSpec((tile,), lambda p, i: (i,)),
        out_shape=jax.ShapeDtypeStruct((B,), _F32),
        scratch_shapes=[
            pltpu.VMEM((B, H1), jnp.bfloat16),
            pltpu.VMEM((B, _D), jnp.bfloat16),
            pltpu.VMEM((2, H1), _F32),
            pltpu.VMEM((B, H2), jnp.bfloat16),
            pltpu.VMEM((2, H2), _F32),
        ],
    )(uc_g, ic_g, w1ut_b, w1it_b, b1r, g1r, be1r, w2t_b, b2r,
      g2r, be2r, wg, wh, bor)
    return out1


# speedup vs baseline: 2.7498x; 1.0466x over previous
import functools

import jax
import jax.numpy as jnp
from jax import lax
from jax.experimental import pallas as pl
from jax.experimental.pallas import tpu as pltpu
from jax.experimental.pallas import tpu_sc as plsc

_D = 64
_NC, _NS = 2, 16
_NW = _NC * _NS
_CHUNK = 128
_TBLK = 8192
_TILE = 2048
_EPS = 1e-5
_F32 = jnp.float32
_HI = lax.Precision.HIGHEST


def _transcat_body(gu, mu, gi, mi, out_ref):
    def pack(g, m):
        lo = lax.bitcast_convert_type(g[...].astype(jnp.bfloat16),
                                      jnp.uint16).astype(jnp.uint32)
        hi = lax.bitcast_convert_type(m[...].astype(jnp.bfloat16),
                                      jnp.uint16).astype(jnp.uint32)
        return lax.bitcast_convert_type(lo | (hi << 16), _F32)

    tu = jnp.transpose(pack(gu, mu), (1, 0))
    ti = jnp.transpose(pack(gi, mi), (1, 0))
    out_ref[...] = jnp.concatenate([tu, ti], axis=1)


def _transcat(gut, mut, git, mit):
    V = gut.shape[1]
    n = -(-V // _TBLK)
    return pl.pallas_call(
        _transcat_body,
        grid=(n,),
        in_specs=[pl.BlockSpec((_D, _TBLK), lambda i: (0, i))] * 4,
        out_specs=pl.BlockSpec((_TBLK, 2 * _D), lambda i: (i, 0)),
        out_shape=jax.ShapeDtypeStruct((V, 2 * _D), _F32),
    )(gut, mut, git, mit)


def _sc_gather_body(uidx, iidx, cat, out_u, out_i,
                    idxu_v, idxi_v, b0, b1, b2, b3, s0, s1, s2, s3):
    rpw = idxu_v.shape[0]
    cpt = rpw // _CHUNK
    wid = lax.axis_index("s") * _NC + lax.axis_index("c")
    base = wid * rpw
    pltpu.sync_copy(uidx.at[pl.ds(base, rpw)], idxu_v)
    pltpu.sync_copy(iidx.at[pl.ds(base, rpw)], idxi_v)

    bufs = (b0, b1, b2, b3)
    sems = (s0, s1, s2, s3)
    units = ([(idxu_v, out_u, j) for j in range(cpt)]
             + [(idxi_v, out_i, j) for j in range(cpt)])
    nbuf = len(bufs)
    copies = {}

    def fire(t):
        idxv, _, j = units[t]
        copies[t] = pltpu.async_copy(
            cat.at[idxv.at[pl.ds(j * _CHUNK, _CHUNK)]],
            bufs[t % nbuf], sems[t % nbuf])

    def drain(t):
        copies[t].wait()
        _, out, j = units[t]
        pltpu.sync_copy(bufs[t % nbuf],
                        out.at[pl.ds(base + j * _CHUNK, _CHUNK)])

    for t in range(len(units)):
        if t >= nbuf:
            drain(t - nbuf)
        fire(t)
    for t in range(len(units) - nbuf, len(units)):
        drain(t)


def _accum_stats(i, y, st_ref):
    ps = jnp.sum(y, axis=0, keepdims=True)
    pq = jnp.sum(y * y, axis=0, keepdims=True)
    part = jnp.concatenate([ps, pq], axis=0)

    @pl.when(i == 0)
    def _():
        st_ref[...] = part

    @pl.when(i > 0)
    def _():
        st_ref[...] += part


def _bn_relu_from_stats(st_ref, n, g, be, y):
    mean = st_ref[0:1, :] * (1.0 / n)
    var = st_ref[1:2, :] * (1.0 / n) - mean * mean
    return jnp.maximum((y - mean) * lax.rsqrt(var + _EPS) * g + be, 0.0)


def _unpack_pair(w):
    u = lax.bitcast_convert_type(w, jnp.uint32)
    g = lax.bitcast_convert_type((u & 0xFFFF).astype(jnp.uint16), jnp.bfloat16)
    m = lax.bitcast_convert_type((u >> 16).astype(jnp.uint16), jnp.bfloat16)
    return g, m


def _dense_body(uc, ic, w1ut, w1it, b1, g1, be1, w2t, b2, g2, be2,
                wg, wh, bo, out_ref, y1_s, xg_s, st1_s, y2_s, st2_s,
                *, n, tile):
    p = pl.program_id(0)
    i = pl.program_id(1)
    rows = pl.ds(i * tile, tile)
    bf16 = jnp.bfloat16

    @pl.when(p == 0)
    def _():
        ug, um = _unpack_pair(uc[...][:, :_D])
        ig, im = _unpack_pair(ic[...][:, _D:])
        y1 = (jnp.dot(um, w1ut[...], preferred_element_type=_F32)
              + jnp.dot(im, w1it[...], preferred_element_type=_F32)
              + b1[...])
        y1_s[rows, :] = y1.astype(bf16)
        xg_s[rows, :] = (ug.astype(_F32) * ig.astype(_F32)).astype(bf16)
        _accum_stats(i, y1, st1_s)

    @pl.when(p == 1)
    def _():
        h1 = _bn_relu_from_stats(st1_s, n, g1[...], be1[...],
                                 y1_s[rows, :].astype(_F32))
        y2 = jnp.dot(h1.astype(bf16), w2t[...],
                     preferred_element_type=_F32) + b2[...]
        y2_s[rows, :] = y2.astype(bf16)
        _accum_stats(i, y2, st2_s)

    @pl.when(p == 2)
    def _():
        h2 = _bn_relu_from_stats(st2_s, n, g2[...], be2[...],
                                 y2_s[rows, :].astype(_F32))
        s = (jnp.sum(xg_s[rows, :].astype(_F32) * wg[...], axis=1)
             + jnp.sum(h2 * wh[...], axis=1) + bo[0, 0])
        out_ref[...] = 1.0 / (1.0 + jnp.exp(-s))


def kernel(user_idx, item_idx, gmf_user, gmf_item, mlp_user, mlp_item,
           W1, b1, g1, be1, W2, b2, g2, be2, Wout, bout):
    B = user_idx.shape[0]
    uidx = user_idx.astype(jnp.int32)
    iidx = item_idx.astype(jnp.int32)

    cat = _transcat(gmf_user.T, mlp_user.T, gmf_item.T, mlp_item.T)

    rpw = B // _NW
    sc_gather = pl.kernel(
        _sc_gather_body,
        out_type=(jax.ShapeDtypeStruct((B, 2 * _D), _F32),) * 2,
        mesh=plsc.VectorSubcoreMesh(core_axis_name="c", subcore_axis_name="s",
                                    num_cores=_NC, num_subcores=_NS),
        compiler_params=pltpu.CompilerParams(use_tc_tiling_on_sc=False),
        scratch_types=(
            [pltpu.VMEM((rpw,), jnp.int32)] * 2
            + [pltpu.VMEM((_CHUNK, 2 * _D), _F32)] * 4
            + [pltpu.SemaphoreType.DMA] * 4
        ),
    )
    uc_g, ic_g = sc_gather(uidx, iidx, cat)

    H1 = W1.shape[0]
    H2 = W2.shape[0]
    tile = _TILE
    nt = B // tile
    w1t = W1.T
    w2t = W2.T

    def full(a):
        return pl.BlockSpec(a.shape, lambda p, i: (0,) * a.ndim)

    gath_spec = pl.BlockSpec((tile, 2 * _D), lambda p, i: (i * (p == 0), 0))

    b1r, g1r, be1r = (v.reshape(1, -1) for v in (b1, g1, be1))
    b2r, g2r, be2r = (v.reshape(1, -1) for v in (b2, g2, be2))

    w1ut_b = w1t[:_D].astype(jnp.bfloat16)
    w1it_b = w1t[_D:].astype(jnp.bfloat16)
    w2t_b = w2t.astype(jnp.bfloat16)
    wg = Wout[:, :_D]
    wh = Wout[:, _D:]
    bor = bout.reshape(1, 1)
    out1 = pl.pallas_call(
        functools.partial(_dense_body, n=float(B), tile=tile),
        grid=(3, nt),
        in_specs=[gath_spec, gath_spec,
                  full(w1ut_b), full(w1it_b), full(b1r),
                  full(g1r), full(be1r), full(w2t_b), full(b2r),
                  full(g2r), full(be2r), full(wg), full(wh), full(bor)],
        out_specs=pl.BlockSpec((tile,), lambda p, i: (i,)),
        out_shape=jax.ShapeDtypeStruct((B,), _F32),
        scratch_shapes=[
            pltpu.VMEM((B, H1), jnp.bfloat16),
            pltpu.VMEM((B, _D), jnp.bfloat16),
            pltpu.VMEM((2, H1), _F32),
            pltpu.VMEM((B, H2), jnp.bfloat16),
            pltpu.VMEM((2, H2), _F32),
        ],
    )(uc_g, ic_g, w1ut_b, w1it_b, b1r, g1r, be1r, w2t_b, b2r,
      g2r, be2r, wg, wh, bor)
    return out1


# speedup vs baseline: 2.8363x; 1.0315x over previous
import functools

import jax
import jax.numpy as jnp
from jax import lax
from jax.experimental import pallas as pl
from jax.experimental.pallas import tpu as pltpu
from jax.experimental.pallas import tpu_sc as plsc

_D = 64
_NC, _NS = 2, 16
_NW = _NC * _NS
_CHUNK = 128
_TBLK = 8192
_TILE = 2048
_EPS = 1e-5
_F32 = jnp.float32
_HI = lax.Precision.HIGHEST


def _transcat_body(gu, mu, gi, mi, out_ref):
    def pack(g, m):
        lo = lax.bitcast_convert_type(g[...].astype(jnp.bfloat16),
                                      jnp.uint16).astype(jnp.uint32)
        hi = lax.bitcast_convert_type(m[...].astype(jnp.bfloat16),
                                      jnp.uint16).astype(jnp.uint32)
        return lax.bitcast_convert_type(lo | (hi << 16), _F32)

    tu = jnp.transpose(pack(gu, mu), (1, 0))
    ti = jnp.transpose(pack(gi, mi), (1, 0))
    out_ref[...] = jnp.concatenate([tu, ti], axis=1)


def _transcat(gut, mut, git, mit):
    V = gut.shape[1]
    n = -(-V // _TBLK)
    return pl.pallas_call(
        _transcat_body,
        grid=(n,),
        in_specs=[pl.BlockSpec((_D, _TBLK), lambda i: (0, i))] * 4,
        out_specs=pl.BlockSpec((_TBLK, 2 * _D), lambda i: (i, 0)),
        out_shape=jax.ShapeDtypeStruct((V, 2 * _D), _F32),
    )(gut, mut, git, mit)


def _sc_gather_body(uidx, iidx, cat, out,
                    idxu_v, idxi_v, b0, b1, b2, b3, s0, s1, s2, s3):
    rpw = idxu_v.shape[0]
    cpt = rpw // _CHUNK
    wid = lax.axis_index("s") * _NC + lax.axis_index("c")
    base = wid * rpw
    pltpu.sync_copy(uidx.at[pl.ds(base, rpw)], idxu_v)
    pltpu.sync_copy(iidx.at[pl.ds(base, rpw)], idxi_v)

    bufs = (b0, b1, b2, b3)
    sems = (s0, s1, s2, s3)
    units = ([(idxu_v, 0, j) for j in range(cpt)]
             + [(idxi_v, _D, j) for j in range(cpt)])
    nbuf = len(bufs)
    copies = {}

    def fire(t):
        idxv, _, j = units[t]
        copies[t] = pltpu.async_copy(
            cat.at[idxv.at[pl.ds(j * _CHUNK, _CHUNK)]],
            bufs[t % nbuf], sems[t % nbuf])

    def drain(t):
        copies[t].wait()
        _, col, j = units[t]
        pltpu.sync_copy(
            bufs[t % nbuf].at[pl.ds(0, _CHUNK), pl.ds(col, _D)],
            out.at[pl.ds(base + j * _CHUNK, _CHUNK), pl.ds(col, _D)])

    for t in range(len(units)):
        if t >= nbuf:
            drain(t - nbuf)
        fire(t)
    for t in range(len(units) - nbuf, len(units)):
        drain(t)


def _accum_stats(i, y, st_ref):
    ps = jnp.sum(y, axis=0, keepdims=True)
    pq = jnp.sum(y * y, axis=0, keepdims=True)
    part = jnp.concatenate([ps, pq], axis=0)

    @pl.when(i == 0)
    def _():
        st_ref[...] = part

    @pl.when(i > 0)
    def _():
        st_ref[...] += part


def _bn_relu_from_stats(st_ref, n, g, be, y):
    mean = st_ref[0:1, :] * (1.0 / n)
    var = st_ref[1:2, :] * (1.0 / n) - mean * mean
    return jnp.maximum((y - mean) * lax.rsqrt(var + _EPS) * g + be, 0.0)


def _unpack_pair(w):
    u = lax.bitcast_convert_type(w, jnp.uint32)
    g = lax.bitcast_convert_type((u & 0xFFFF).astype(jnp.uint16), jnp.bfloat16)
    m = lax.bitcast_convert_type((u >> 16).astype(jnp.uint16), jnp.bfloat16)
    return g, m


def _dense_body(gth, w1ut, w1it, b1, g1, be1, w2t, b2, g2, be2,
                wg, wh, bo, out_ref, y1_s, xg_s, st1_s, y2_s, st2_s,
                *, n, tile):
    p = pl.program_id(0)
    i = pl.program_id(1)
    rows = pl.ds(i * tile, tile)
    bf16 = jnp.bfloat16

    @pl.when(p == 0)
    def _():
        gv = gth[...]
        ug, um = _unpack_pair(gv[:, :_D])
        ig, im = _unpack_pair(gv[:, _D:])
        y1 = (jnp.dot(um, w1ut[...], preferred_element_type=_F32)
              + jnp.dot(im, w1it[...], preferred_element_type=_F32)
              + b1[...])
        y1_s[rows, :] = y1.astype(bf16)
        xg_s[rows, :] = (ug.astype(_F32) * ig.astype(_F32)).astype(bf16)
        _accum_stats(i, y1, st1_s)

    @pl.when(p == 1)
    def _():
        h1 = _bn_relu_from_stats(st1_s, n, g1[...], be1[...],
                                 y1_s[rows, :].astype(_F32))
        y2 = jnp.dot(h1.astype(bf16), w2t[...],
                     preferred_element_type=_F32) + b2[...]
        y2_s[rows, :] = y2.astype(bf16)
        _accum_stats(i, y2, st2_s)

    @pl.when(p == 2)
    def _():
        h2 = _bn_relu_from_stats(st2_s, n, g2[...], be2[...],
                                 y2_s[rows, :].astype(_F32))
        s = (jnp.sum(xg_s[rows, :].astype(_F32) * wg[...], axis=1)
             + jnp.sum(h2 * wh[...], axis=1) + bo[0, 0])
        out_ref[...] = 1.0 / (1.0 + jnp.exp(-s))


def kernel(user_idx, item_idx, gmf_user, gmf_item, mlp_user, mlp_item,
           W1, b1, g1, be1, W2, b2, g2, be2, Wout, bout):
    B = user_idx.shape[0]
    uidx = user_idx.astype(jnp.int32)
    iidx = item_idx.astype(jnp.int32)

    cat = _transcat(gmf_user.T, mlp_user.T, gmf_item.T, mlp_item.T)

    rpw = B // _NW
    sc_gather = pl.kernel(
        _sc_gather_body,
        out_type=jax.ShapeDtypeStruct((B, 2 * _D), _F32),
        mesh=plsc.VectorSubcoreMesh(core_axis_name="c", subcore_axis_name="s",
                                    num_cores=_NC, num_subcores=_NS),
        compiler_params=pltpu.CompilerParams(use_tc_tiling_on_sc=False),
        scratch_types=(
            [pltpu.VMEM((rpw,), jnp.int32)] * 2
            + [pltpu.VMEM((_CHUNK, 2 * _D), _F32)] * 4
            + [pltpu.SemaphoreType.DMA] * 4
        ),
    )
    gath = sc_gather(uidx, iidx, cat)

    H1 = W1.shape[0]
    H2 = W2.shape[0]
    tile = _TILE
    nt = B // tile
    w1t = W1.T
    w2t = W2.T

    def full(a):
        return pl.BlockSpec(a.shape, lambda p, i: (0,) * a.ndim)

    gath_spec = pl.BlockSpec((tile, 2 * _D), lambda p, i: (i * (p == 0), 0))

    b1r, g1r, be1r = (v.reshape(1, -1) for v in (b1, g1, be1))
    b2r, g2r, be2r = (v.reshape(1, -1) for v in (b2, g2, be2))

    w1ut_b = w1t[:_D].astype(jnp.bfloat16)
    w1it_b = w1t[_D:].astype(jnp.bfloat16)
    w2t_b = w2t.astype(jnp.bfloat16)
    wg = Wout[:, :_D]
    wh = Wout[:, _D:]
    bor = bout.reshape(1, 1)
    out1 = pl.pallas_call(
        functools.partial(_dense_body, n=float(B), tile=tile),
        grid=(3, nt),
        in_specs=[gath_spec,
                  full(w1ut_b), full(w1it_b), full(b1r),
                  full(g1r), full(be1r), full(w2t_b), full(b2r),
                  full(g2r), full(be2r), full(wg), full(wh), full(bor)],
        out_specs=pl.BlockSpec((tile,), lambda p, i: (i,)),
        out_shape=jax.ShapeDtypeStruct((B,), _F32),
        scratch_shapes=[
            pltpu.VMEM((B, H1), jnp.bfloat16),
            pltpu.VMEM((B, _D), jnp.bfloat16),
            pltpu.VMEM((2, H1), _F32),
            pltpu.VMEM((B, H2), jnp.bfloat16),
            pltpu.VMEM((2, H2), _F32),
        ],
    )(gath, w1ut_b, w1it_b, b1r, g1r, be1r, w2t_b, b2r,
      g2r, be2r, wg, wh, bor)
    return out1


# speedup vs baseline: 2.9059x; 1.0246x over previous
import functools

import jax
import jax.numpy as jnp
from jax import lax
from jax.experimental import pallas as pl
from jax.experimental.pallas import tpu as pltpu
from jax.experimental.pallas import tpu_sc as plsc

_D = 64
_NC, _NS = 2, 16
_NW = _NC * _NS
_CHUNK = 128
_TBLK = 8192
_TILE = 4096
_EPS = 1e-5
_F32 = jnp.float32
_HI = lax.Precision.HIGHEST


def _transcat_body(gu, mu, gi, mi, out_ref):
    def pack(g, m):
        lo = lax.bitcast_convert_type(g[...].astype(jnp.bfloat16),
                                      jnp.uint16).astype(jnp.uint32)
        hi = lax.bitcast_convert_type(m[...].astype(jnp.bfloat16),
                                      jnp.uint16).astype(jnp.uint32)
        return lax.bitcast_convert_type(lo | (hi << 16), _F32)

    tu = jnp.transpose(pack(gu, mu), (1, 0))
    ti = jnp.transpose(pack(gi, mi), (1, 0))
    out_ref[...] = jnp.concatenate([tu, ti], axis=1)


def _transcat(gut, mut, git, mit):
    V = gut.shape[1]
    n = -(-V // _TBLK)
    return pl.pallas_call(
        _transcat_body,
        grid=(n,),
        in_specs=[pl.BlockSpec((_D, _TBLK), lambda i: (0, i))] * 4,
        out_specs=pl.BlockSpec((_TBLK, 2 * _D), lambda i: (i, 0)),
        out_shape=jax.ShapeDtypeStruct((V, 2 * _D), _F32),
    )(gut, mut, git, mit)


def _sc_gather_body(uidx, iidx, cat, out,
                    idxu_v, idxi_v, b0, b1, b2, b3, s0, s1, s2, s3):
    rpw = idxu_v.shape[0]
    cpt = rpw // _CHUNK
    wid = lax.axis_index("s") * _NC + lax.axis_index("c")
    base = wid * rpw
    pltpu.sync_copy(uidx.at[pl.ds(base, rpw)], idxu_v)
    pltpu.sync_copy(iidx.at[pl.ds(base, rpw)], idxi_v)

    bufs = (b0, b1, b2, b3)
    sems = (s0, s1, s2, s3)
    units = ([(idxu_v, 0, j) for j in range(cpt)]
             + [(idxi_v, _D, j) for j in range(cpt)])
    nbuf = len(bufs)
    copies = {}

    def fire(t):
        idxv, _, j = units[t]
        copies[t] = pltpu.async_copy(
            cat.at[idxv.at[pl.ds(j * _CHUNK, _CHUNK)]],
            bufs[t % nbuf], sems[t % nbuf])

    def drain(t):
        copies[t].wait()
        _, col, j = units[t]
        pltpu.sync_copy(
            bufs[t % nbuf].at[pl.ds(0, _CHUNK), pl.ds(col, _D)],
            out.at[pl.ds(base + j * _CHUNK, _CHUNK), pl.ds(col, _D)])

    for t in range(len(units)):
        if t >= nbuf:
            drain(t - nbuf)
        fire(t)
    for t in range(len(units) - nbuf, len(units)):
        drain(t)


def _accum_stats(i, y, st_ref):
    ps = jnp.sum(y, axis=0, keepdims=True)
    pq = jnp.sum(y * y, axis=0, keepdims=True)
    part = jnp.concatenate([ps, pq], axis=0)

    @pl.when(i == 0)
    def _():
        st_ref[...] = part

    @pl.when(i > 0)
    def _():
        st_ref[...] += part


def _bn_relu_from_stats(st_ref, n, g, be, y):
    mean = st_ref[0:1, :] * (1.0 / n)
    var = st_ref[1:2, :] * (1.0 / n) - mean * mean
    return jnp.maximum((y - mean) * lax.rsqrt(var + _EPS) * g + be, 0.0)


def _unpack_pair(w):
    u = lax.bitcast_convert_type(w, jnp.uint32)
    g = lax.bitcast_convert_type((u & 0xFFFF).astype(jnp.uint16), jnp.bfloat16)
    m = lax.bitcast_convert_type((u >> 16).astype(jnp.uint16), jnp.bfloat16)
    return g, m


def _dense_body(gth, w1ut, w1it, b1, g1, be1, w2t, b2, g2, be2,
                wg, wh, bo, out_ref, y1_s, xg_s, st1_s, y2_s, st2_s,
                *, n, tile):
    p = pl.program_id(0)
    i = pl.program_id(1)
    rows = pl.ds(i * tile, tile)
    bf16 = jnp.bfloat16

    @pl.when(p == 0)
    def _():
        gv = gth[...]
        ug, um = _unpack_pair(gv[:, :_D])
        ig, im = _unpack_pair(gv[:, _D:])
        y1 = (jnp.dot(um, w1ut[...], preferred_element_type=_F32)
              + jnp.dot(im, w1it[...], preferred_element_type=_F32)
              + b1[...])
        y1_s[rows, :] = y1.astype(bf16)
        xg_s[rows, :] = (ug.astype(_F32) * ig.astype(_F32)).astype(bf16)
        _accum_stats(i, y1, st1_s)

    @pl.when(p == 1)
    def _():
        h1 = _bn_relu_from_stats(st1_s, n, g1[...], be1[...],
                                 y1_s[rows, :].astype(_F32))
        y2 = jnp.dot(h1.astype(bf16), w2t[...],
                     preferred_element_type=_F32) + b2[...]
        y2_s[rows, :] = y2.astype(bf16)
        _accum_stats(i, y2, st2_s)

    @pl.when(p == 2)
    def _():
        h2 = _bn_relu_from_stats(st2_s, n, g2[...], be2[...],
                                 y2_s[rows, :].astype(_F32))
        s = (jnp.sum(xg_s[rows, :].astype(_F32) * wg[...], axis=1)
             + jnp.sum(h2 * wh[...], axis=1) + bo[0, 0])
        out_ref[...] = 1.0 / (1.0 + jnp.exp(-s))


def kernel(user_idx, item_idx, gmf_user, gmf_item, mlp_user, mlp_item,
           W1, b1, g1, be1, W2, b2, g2, be2, Wout, bout):
    B = user_idx.shape[0]
    uidx = user_idx.astype(jnp.int32)
    iidx = item_idx.astype(jnp.int32)

    cat = _transcat(gmf_user.T, mlp_user.T, gmf_item.T, mlp_item.T)

    rpw = B // _NW
    sc_gather = pl.kernel(
        _sc_gather_body,
        out_type=jax.ShapeDtypeStruct((B, 2 * _D), _F32),
        mesh=plsc.VectorSubcoreMesh(core_axis_name="c", subcore_axis_name="s",
                                    num_cores=_NC, num_subcores=_NS),
        compiler_params=pltpu.CompilerParams(use_tc_tiling_on_sc=False),
        scratch_types=(
            [pltpu.VMEM((rpw,), jnp.int32)] * 2
            + [pltpu.VMEM((_CHUNK, 2 * _D), _F32)] * 4
            + [pltpu.SemaphoreType.DMA] * 4
        ),
    )
    gath = sc_gather(uidx, iidx, cat)

    H1 = W1.shape[0]
    H2 = W2.shape[0]
    tile = _TILE
    nt = B // tile
    w1t = W1.T
    w2t = W2.T

    def full(a):
        return pl.BlockSpec(a.shape, lambda p, i: (0,) * a.ndim)

    gath_spec = pl.BlockSpec((tile, 2 * _D), lambda p, i: (i * (p == 0), 0))

    b1r, g1r, be1r = (v.reshape(1, -1) for v in (b1, g1, be1))
    b2r, g2r, be2r = (v.reshape(1, -1) for v in (b2, g2, be2))

    w1ut_b = w1t[:_D].astype(jnp.bfloat16)
    w1it_b = w1t[_D:].astype(jnp.bfloat16)
    w2t_b = w2t.astype(jnp.bfloat16)
    wg = Wout[:, :_D]
    wh = Wout[:, _D:]
    bor = bout.reshape(1, 1)
    out1 = pl.pallas_call(
        functools.partial(_dense_body, n=float(B), tile=tile),
        grid=(3, nt),
        in_specs=[gath_spec,
                  full(w1ut_b), full(w1it_b), full(b1r),
                  full(g1r), full(be1r), full(w2t_b), full(b2r),
                  full(g2r), full(be2r), full(wg), full(wh), full(bor)],
        out_specs=pl.BlockSpec((tile,), lambda p, i: (i,)),
        out_shape=jax.ShapeDtypeStruct((B,), _F32),
        scratch_shapes=[
            pltpu.VMEM((B, H1), jnp.bfloat16),
            pltpu.VMEM((B, _D), jnp.bfloat16),
            pltpu.VMEM((2, H1), _F32),
            pltpu.VMEM((B, H2), jnp.bfloat16),
            pltpu.VMEM((2, H2), _F32),
        ],
    )(gath, w1ut_b, w1it_b, b1r, g1r, be1r, w2t_b, b2r,
      g2r, be2r, wg, wh, bor)
    return out1
